# Initial kernel scaffold; baseline (speedup 1.0000x reference)
import jax
import jax.numpy as jnp
from jax.experimental import pallas as pl


def kernel(cell, x, z, struct_size, edge_index, edge_attr, emb, W1_0, b1_0, W2_0, b2_0, W1_1, b1_1, W2_1, b2_1, Wu1, bu1, Wu2, bu2, Wa1, ba1, Wa2, ba2):
    raise NotImplementedError("write your pallas kernel here")


# speedup vs baseline: 5.4464x; 5.4464x over previous
import functools

import jax
import jax.numpy as jnp
from jax import lax
from jax.experimental import pallas as pl
from jax.experimental.pallas import tpu as pltpu
from jax.experimental.pallas import tpu_sc as plsc

NC = 2
NS = 16
NW = NC * NS
CHUNK = 80

_HIGH = jax.lax.Precision.HIGHEST


def _dot(a, b):
    return jax.lax.dot_general(a, b, (((1,), (0,)), ((), ())),
                               precision=_HIGH,
                               preferred_element_type=jnp.float32)


def _sc_gather(table, idx):
    M = idx.shape[0]
    D = table.shape[1]
    per_w = M // NW
    n_chunks = per_w // CHUNK
    mesh = plsc.VectorSubcoreMesh(core_axis_name="c", subcore_axis_name="s",
                                  num_cores=NC, num_subcores=NS)

    @functools.partial(
        pl.kernel, mesh=mesh,
        out_type=jax.ShapeDtypeStruct((M, D), jnp.float32),
        compiler_params=pltpu.CompilerParams(
            use_tc_tiling_on_sc=(D % 128 == 0)),
        scratch_types=[
            pltpu.VMEM((CHUNK,), jnp.int32),
            pltpu.VMEM((CHUNK, D), jnp.float32),
            pltpu.SemaphoreType.DMA,
        ],
    )
    def k(table_hbm, idx_hbm, out_hbm, idx_v, rows_v, sem):
        wid = lax.axis_index("s") * NC + lax.axis_index("c")

        def body(t, _):
            base = wid * per_w + t * CHUNK
            pltpu.sync_copy(idx_hbm.at[pl.ds(base, CHUNK)], idx_v)
            pltpu.async_copy(table_hbm.at[idx_v], rows_v, sem).wait()
            pltpu.sync_copy(rows_v, out_hbm.at[pl.ds(base, CHUNK)])
            return 0

        lax.fori_loop(0, n_chunks, body, 0)

    return k(table, idx)


def _sc_scatter_add(m, dst, n_rows):
    E, D = m.shape
    per_w = E // NW
    n_chunks = per_w // CHUNK
    n_slabs = max(k for k in range(1, NS + 1)
                  if n_rows % k == 0 and (n_rows // k) % 8 == 0)
    slab = n_rows // n_slabs
    mesh = plsc.VectorSubcoreMesh(core_axis_name="c", subcore_axis_name="s",
                                  num_cores=NC, num_subcores=NS)
    zeros = jnp.zeros((n_rows, D), jnp.float32)

    @functools.partial(
        pl.kernel, mesh=mesh,
        out_type=jax.ShapeDtypeStruct((NC, n_rows, D), jnp.float32),
        scratch_types=[
            pltpu.MemorySpace.VMEM_SHARED((n_rows, D), jnp.float32),
            pltpu.VMEM((CHUNK,), jnp.int32),
            pltpu.VMEM((CHUNK, D), jnp.float32),
        ],
    )
    def k(m_hbm, dst_hbm, z_hbm, parts_hbm, agg_sh, idx_v, rows_v):
        cid = lax.axis_index("c")
        sid = lax.axis_index("s")
        wid = cid * NS + sid
        row0 = sid * slab

        @pl.when(sid < n_slabs)
        def _():
            pltpu.sync_copy(z_hbm.at[pl.ds(row0, slab)],
                            agg_sh.at[pl.ds(row0, slab)])
        plsc.subcore_barrier()

        def body(t, _):
            base = wid * per_w + t * CHUNK
            pltpu.sync_copy(dst_hbm.at[pl.ds(base, CHUNK)], idx_v)
            pltpu.sync_copy(m_hbm.at[pl.ds(base, CHUNK)], rows_v)
            pltpu.sync_copy(rows_v, agg_sh.at[idx_v], add=True)
            return 0

        lax.fori_loop(0, n_chunks, body, 0)
        plsc.subcore_barrier()

        @pl.when(sid < n_slabs)
        def _():
            pltpu.sync_copy(agg_sh.at[pl.ds(row0, slab)],
                            parts_hbm.at[cid, pl.ds(row0, slab)])

    return k(m, dst, zeros)


def _col(a, j):
    return a[:, j:j + 1]


def _tc_prep(xg, sh, ends, E, Eb):
    nb = E // Eb

    def fn(xs_ref, xd_ref, sh_ref, ends_ref, ef_ref):
        xs = xs_ref[...]
        xd = xd_ref[...]
        s = sh_ref[...]
        d = (xd - jnp.floor(xd)) - (xs - jnp.floor(xs)) + s
        src_f = _col(s, 3)
        bf = jnp.sum((src_f >= ends_ref[...]).astype(jnp.float32), axis=1,
                     keepdims=True)
        ef_ref[...] = jnp.concatenate(
            [d[:, 0:3], bf, jnp.zeros((xs.shape[0], 12), jnp.float32)], axis=1)

    return pl.pallas_call(
        fn,
        grid=(nb,),
        in_specs=[
            pl.BlockSpec((Eb, 16), lambda i: (i, 0)),
            pl.BlockSpec((Eb, 16), lambda i: (nb + i, 0)),
            pl.BlockSpec((Eb, 16), lambda i: (i, 0)),
            pl.BlockSpec((1, 16), lambda i: (0, 0)),
        ],
        out_specs=pl.BlockSpec((Eb, 16), lambda i: (i, 0)),
        out_shape=jax.ShapeDtypeStruct((E, 16), jnp.float32),
    )(xg, xg, sh, ends)


def _vec_cols(ef, rho16):
    bf = _col(ef, 3)
    iot = lax.broadcasted_iota(jnp.int32, (ef.shape[0], 16), 1)
    onehot = (bf == iot.astype(jnp.float32))
    P = _dot(onehot.astype(jnp.float32), rho16)
    v = []
    for j in range(3):
        v.append(_col(ef, 0) * _col(P, 0 + j) + _col(ef, 1) * _col(P, 3 + j)
                 + _col(ef, 2) * _col(P, 6 + j))
    return v, onehot


def _tc_message(g, ef, rho16, w1c, E, Eb, a_off):
    nb = E // Eb

    def fn(as_ref, bd_ref, ef_ref, rho_ref, w1c_ref, m_ref):
        ef_b = ef_ref[...]
        v, _ = _vec_cols(ef_b, rho_ref[...])
        vw = (v[0] * w1c_ref[0:1, :] + v[1] * w1c_ref[1:2, :]
              + v[2] * w1c_ref[2:3, :])
        m_ref[...] = jnp.maximum(as_ref[...] + bd_ref[...] + vw, 0.0)

    return pl.pallas_call(
        fn,
        grid=(nb,),
        in_specs=[
            pl.BlockSpec((Eb, 128), lambda i: (a_off + i, 0)),
            pl.BlockSpec((Eb, 128), lambda i: (a_off + nb + i, 0)),
            pl.BlockSpec((Eb, 16), lambda i: (i, 0)),
            pl.BlockSpec((16, 16), lambda i: (0, 0)),
            pl.BlockSpec((8, 128), lambda i: (0, 0)),
        ],
        out_specs=pl.BlockSpec((Eb, 128), lambda i: (i, 0)),
        out_shape=jax.ShapeDtypeStruct((E, 128), jnp.float32),
    )(g, g, ef, rho16, w1c)


def _tc_update(parts, h, W2, b2p, ab_ws, N, Nb):
    nb = N // Nb
    n_ab = len(ab_ws)

    def fn(*refs):
        parts_ref, h_ref = refs[0], refs[1]
        W2_ref, b2_ref = refs[2], refs[3]
        wrefs = refs[4:4 + 3 * n_ab]
        h_out = refs[4 + 3 * n_ab]
        ab_out = refs[5 + 3 * n_ab]
        agg = parts_ref[0] + parts_ref[1]
        hn = h_ref[...] + jnp.maximum(_dot(agg, W2_ref[...])
                                      + b2_ref[0:1, :], 0.0)
        h_out[...] = hn
        for j in range(n_ab):
            wa, wb, bp = wrefs[3 * j], wrefs[3 * j + 1], wrefs[3 * j + 2]
            ab_out[2 * j] = _dot(hn, wa[...]) + bp[0:1, :]
            ab_out[2 * j + 1] = _dot(hn, wb[...])

    in_specs = [
        pl.BlockSpec((2, Nb, 128), lambda i: (0, i, 0)),
        pl.BlockSpec((Nb, 128), lambda i: (i, 0)),
        pl.BlockSpec((128, 128), lambda i: (0, 0)),
        pl.BlockSpec((8, 128), lambda i: (0, 0)),
    ]
    args = [parts, h, W2, b2p]
    for (wa, wb, bp) in ab_ws:
        in_specs += [pl.BlockSpec((128, 128), lambda i: (0, 0)),
                     pl.BlockSpec((128, 128), lambda i: (0, 0)),
                     pl.BlockSpec((8, 128), lambda i: (0, 0))]
        args += [wa, wb, bp]
    return pl.pallas_call(
        fn,
        grid=(nb,),
        in_specs=in_specs,
        out_specs=[pl.BlockSpec((Nb, 128), lambda i: (i, 0)),
                   pl.BlockSpec((2 * n_ab, Nb, 128), lambda i: (0, i, 0))],
        out_shape=[jax.ShapeDtypeStruct((N, 128), jnp.float32),
                   jax.ShapeDtypeStruct((2 * n_ab, N, 128), jnp.float32)],
    )(*args)


def _tc_ab0(h, wa, wb, bp, N, Nb):
    nb = N // Nb

    def fn(h_ref, wa_ref, wb_ref, bp_ref, ab_ref):
        hb = h_ref[...]
        ab_ref[0] = _dot(hb, wa_ref[...]) + bp_ref[0:1, :]
        ab_ref[1] = _dot(hb, wb_ref[...])

    return pl.pallas_call(
        fn,
        grid=(nb,),
        in_specs=[
            pl.BlockSpec((Nb, 128), lambda i: (i, 0)),
            pl.BlockSpec((128, 128), lambda i: (0, 0)),
            pl.BlockSpec((128, 128), lambda i: (0, 0)),
            pl.BlockSpec((8, 128), lambda i: (0, 0)),
        ],
        out_specs=pl.BlockSpec((2, Nb, 128), lambda i: (0, i, 0)),
        out_shape=jax.ShapeDtypeStruct((2, N, 128), jnp.float32),
    )(h, wa, wb, bp)


def _tc_actions(g, ef, rho16, wa1c, wa2p, ba2p, E, Eb, a_off):
    nb = E // Eb

    def fn(as_ref, bd_ref, ef_ref, rho_ref, wc_ref, wa2_ref, ba2_ref, acc_ref):
        i = pl.program_id(0)
        ef_b = ef_ref[...]
        v, onehot = _vec_cols(ef_b, rho_ref[...])
        vw = (v[0] * wc_ref[0:1, :] + v[1] * wc_ref[1:2, :]
              + v[2] * wc_ref[2:3, :])
        e = jnp.maximum(as_ref[...] + bd_ref[...] + vw, 0.0)
        w8 = _dot(e, wa2_ref[...])
        w = jnp.tanh(_col(w8, 0) + ba2_ref[0:1, 0:1])
        nrm = jnp.sqrt(v[0] * v[0] + v[1] * v[1] + v[2] * v[2]) + 1e-8
        u = [vj / nrm for vj in v]
        cols = []
        for a in range(3):
            for b in range(3):
                cols.append(w * u[a] * u[b])
        cols.append(jnp.ones_like(w))
        g16 = jnp.concatenate(
            cols + [jnp.zeros((ef_b.shape[0], 6), jnp.float32)], axis=1)
        seg = jax.lax.dot_general(onehot.astype(jnp.float32), g16,
                                  (((0,), (0,)), ((), ())),
                                  precision=_HIGH,
                                  preferred_element_type=jnp.float32)

        @pl.when(i == 0)
        def _():
            acc_ref[...] = jnp.zeros_like(acc_ref)

        acc_ref[...] += seg

    return pl.pallas_call(
        fn,
        grid=(nb,),
        in_specs=[
            pl.BlockSpec((Eb, 128), lambda i: (a_off + i, 0)),
            pl.BlockSpec((Eb, 128), lambda i: (a_off + nb + i, 0)),
            pl.BlockSpec((Eb, 16), lambda i: (i, 0)),
            pl.BlockSpec((16, 16), lambda i: (0, 0)),
            pl.BlockSpec((8, 128), lambda i: (0, 0)),
            pl.BlockSpec((128, 8), lambda i: (0, 0)),
            pl.BlockSpec((8, 8), lambda i: (0, 0)),
        ],
        out_specs=pl.BlockSpec((16, 16), lambda i: (0, 0)),
        out_shape=jax.ShapeDtypeStruct((16, 16), jnp.float32),
    )(g, g, ef, rho16, wa1c, wa2p, ba2p)


def _tc_compose(acc, ar_prev, cell16, limit):
    def fn(acc_ref, ar_ref, cell_ref, arn_ref, rho_ref):
        acc_b = acc_ref[...]
        ar = ar_ref[...]
        cl = cell_ref[...]
        cnt = _col(acc_b, 9) + 1e-8
        act = [None] * 9
        for i in range(3):
            for j in range(3):
                act[3 * i + j] = (limit * _col(acc_b, 3 * i + j) / cnt
                                  + (1.0 if i == j else 0.0))
        arn = [None] * 9
        for i in range(3):
            for k in range(3):
                arn[3 * i + k] = (act[3 * i] * _col(ar, k)
                                  + act[3 * i + 1] * _col(ar, 3 + k)
                                  + act[3 * i + 2] * _col(ar, 6 + k))
        rho = [None] * 9
        for i in range(3):
            for k in range(3):
                rho[3 * i + k] = (arn[3 * i] * _col(cl, k)
                                  + arn[3 * i + 1] * _col(cl, 3 + k)
                                  + arn[3 * i + 2] * _col(cl, 6 + k))
        z7 = jnp.zeros((16, 7), jnp.float32)
        arn_ref[...] = jnp.concatenate(arn + [z7], axis=1)
        rho_ref[...] = jnp.concatenate(rho + [z7], axis=1)

    return pl.pallas_call(
        fn,
        grid=(1,),
        in_specs=[pl.BlockSpec((16, 16), lambda i: (0, 0))] * 3,
        out_specs=[pl.BlockSpec((16, 16), lambda i: (0, 0))] * 2,
        out_shape=[jax.ShapeDtypeStruct((16, 16), jnp.float32)] * 2,
    )(acc, ar_prev, cell16)


def kernel(cell, x, z, struct_size, edge_index, edge_attr, emb,
           W1_0, b1_0, W2_0, b2_0, W1_1, b1_1, W2_1, b2_1,
           Wu1, bu1, Wu2, bu2, Wa1, ba1, Wa2, ba2):
    N = x.shape[0]
    E = edge_index.shape[1]
    B = cell.shape[0]
    LIMIT = 0.1
    Eb = 2000
    Nb = 1000

    src = edge_index[0].astype(jnp.int32)
    dst = edge_index[1].astype(jnp.int32)
    idx2 = jnp.concatenate([src, dst + N])
    idx4 = jnp.concatenate([src, dst + N, src + 2 * N, dst + 3 * N])

    xpad = jnp.pad(x.astype(jnp.float32), ((0, 0), (0, 13)))
    xpad2 = jnp.concatenate([xpad, xpad], axis=0)

    sh = jnp.concatenate([
        (edge_attr - 1).astype(jnp.float32),
        src.astype(jnp.float32)[:, None],
        jnp.zeros((E, 12), jnp.float32)], axis=1)

    ends = jnp.cumsum(struct_size).astype(jnp.float32)[None, :]

    zp_len = ((N + NW * CHUNK - 1) // (NW * CHUNK)) * (NW * CHUNK)
    zpad = jnp.pad(z.astype(jnp.int32), (0, zp_len - N))

    def wsplit(W, b):
        wa = W[:128]
        wb = W[128:256]
        wc = jnp.pad(W[256:259], ((0, 5), (0, 0)))
        bp = jnp.broadcast_to(b[None, :], (8, 128))
        return wa, wb, wc, bp

    W1a_0, W1b_0, W1c_0, b1p_0 = wsplit(W1_0, b1_0)
    W1a_1, W1b_1, W1c_1, b1p_1 = wsplit(W1_1, b1_1)
    Wua, Wub, Wuc, bup = wsplit(Wu1, bu1)
    Waa, Wab, Wac, bap = wsplit(Wa1, ba1)
    Wa2p = jnp.pad(Wa2, ((0, 0), (0, 7)))
    ba2p = jnp.broadcast_to(ba2[None, :], (8, 8))
    b2p_0 = jnp.broadcast_to(b2_0[None, :], (8, 128))
    b2p_1 = jnp.broadcast_to(b2_1[None, :], (8, 128))
    bu2p = jnp.broadcast_to(bu2[None, :], (8, 128))

    cell16 = jnp.pad(cell.reshape(B, 9), ((0, 0), (0, 7)))
    eye16 = jnp.pad(jnp.eye(3, dtype=jnp.float32).reshape(1, 9),
                    ((0, 0), (0, 7)))
    eye16 = jnp.broadcast_to(eye16, (16, 16))

    xg = _sc_gather(xpad2, idx2)
    ef = _tc_prep(xg, sh, ends, E, Eb)
    h0 = _sc_gather(emb, zpad)[:N]

    ab = _tc_ab0(h0, W1a_0, W1b_0, b1p_0, N, Nb)
    g = _sc_gather(ab.reshape(2 * N, 128), idx2)
    m = _tc_message(g, ef, cell16, W1c_0, E, Eb, 0)
    parts = _sc_scatter_add(m, dst, N)
    h1, ab = _tc_update(parts, h0, W2_0, b2p_0,
                        [(W1a_1, W1b_1, b1p_1)], N, Nb)

    g = _sc_gather(ab.reshape(2 * N, 128), idx2)
    m = _tc_message(g, ef, cell16, W1c_1, E, Eb, 0)
    parts = _sc_scatter_add(m, dst, N)
    h2, ab = _tc_update(parts, h1, W2_1, b2p_1, [(Wua, Wub, bup)], N, Nb)

    g = _sc_gather(ab.reshape(2 * N, 128), idx2)
    m = _tc_message(g, ef, cell16, Wuc, E, Eb, 0)
    parts = _sc_scatter_add(m, dst, N)
    h3, ab4 = _tc_update(parts, h2, Wu2, bu2p,
                         [(Wua, Wub, bup), (Waa, Wab, bap)], N, Nb)

    g4 = _sc_gather(ab4.reshape(4 * N, 128), idx4)
    nb_e = E // Eb
    acc0 = _tc_actions(g4, ef, cell16, Wac, Wa2p, ba2p, E, Eb, 2 * nb_e)
    ar0, rho1 = _tc_compose(acc0, eye16, cell16, LIMIT)

    m = _tc_message(g4, ef, rho1, Wuc, E, Eb, 0)
    parts = _sc_scatter_add(m, dst, N)
    h4, ab = _tc_update(parts, h3, Wu2, bu2p, [(Waa, Wab, bap)], N, Nb)

    g = _sc_gather(ab.reshape(2 * N, 128), idx2)
    acc1 = _tc_actions(g, ef, rho1, Wac, Wa2p, ba2p, E, Eb, 0)
    ar1, rho2 = _tc_compose(acc1, ar0, cell16, LIMIT)

    rho1m = rho1[:, :9].reshape(B, 3, 3)
    rho2m = rho2[:, :9].reshape(B, 3, 3)
    ar0m = ar0[:, :9].reshape(B, 3, 3)
    ar1m = ar1[:, :9].reshape(B, 3, 3)
    return (rho2m, jnp.stack([rho1m, rho2m]), jnp.stack([ar0m, ar1m]))


# speedup vs baseline: 7.0599x; 1.2962x over previous
import functools

import jax
import jax.numpy as jnp
from jax import lax
from jax.experimental import pallas as pl
from jax.experimental.pallas import tpu as pltpu
from jax.experimental.pallas import tpu_sc as plsc

NC = 2
NS = 16
NW = NC * NS
CHUNK = 80

_HIGH = jax.lax.Precision.HIGHEST


def _dot(a, b):
    return jax.lax.dot_general(a, b, (((1,), (0,)), ((), ())),
                               precision=_HIGH,
                               preferred_element_type=jnp.float32)


def _sc_gather(table, idx):
    M = idx.shape[0]
    D = table.shape[1]
    per_w = M // NW
    n_chunks = per_w // CHUNK
    K = max(k for k in range(1, 7)
            if n_chunks % k == 0 and (n_chunks // k) % 2 == 0)
    n_grp = n_chunks // K
    grp_rows = K * CHUNK
    mesh = plsc.VectorSubcoreMesh(core_axis_name="c", subcore_axis_name="s",
                                  num_cores=NC, num_subcores=NS)

    @functools.partial(
        pl.kernel, mesh=mesh,
        out_type=jax.ShapeDtypeStruct((M, D), jnp.float32),
        compiler_params=pltpu.CompilerParams(
            use_tc_tiling_on_sc=(D % 128 == 0)),
        scratch_types=[
            pltpu.VMEM((grp_rows,), jnp.int32),
            pltpu.VMEM((grp_rows,), jnp.int32),
            pltpu.VMEM((grp_rows, D), jnp.float32),
            pltpu.VMEM((grp_rows, D), jnp.float32),
            pltpu.SemaphoreType.DMA,
            pltpu.SemaphoreType.DMA,
            pltpu.SemaphoreType.DMA,
            pltpu.SemaphoreType.DMA,
            pltpu.SemaphoreType.DMA,
            pltpu.SemaphoreType.DMA,
        ],
    )
    def k(table_hbm, idx_hbm, out_hbm, idx0, idx1, rows0, rows1,
          sg0, sg1, so0, so1, si0, si1):
        wid = lax.axis_index("s") * NC + lax.axis_index("c")
        idxb = (idx0, idx1)
        rows = (rows0, rows1)
        sg = (sg0, sg1)
        so = (so0, so1)
        si = (si0, si1)

        def idx_copy(g, p):
            return pltpu.make_async_copy(
                idx_hbm.at[pl.ds(wid * per_w + g * grp_rows, grp_rows)],
                idxb[p], si[p])

        def fire(g, p):
            for b in range(K):
                pltpu.async_copy(
                    table_hbm.at[idxb[p].at[pl.ds(b * CHUNK, CHUNK)]],
                    rows[p].at[pl.ds(b * CHUNK, CHUNK)], sg[p])

        def drain_gather(g, p):
            for b in range(K):
                pltpu.make_async_copy(
                    table_hbm.at[idxb[p].at[pl.ds(b * CHUNK, CHUNK)]],
                    rows[p].at[pl.ds(b * CHUNK, CHUNK)], sg[p]).wait()

        def out_copy(g, p):
            return pltpu.make_async_copy(
                rows[p],
                out_hbm.at[pl.ds(wid * per_w + g * grp_rows, grp_rows)],
                so[p])

        idx_copy(0, 0).start()
        idx_copy(0, 0).wait()
        fire(0, 0)
        idx_copy(1, 1).start()
        idx_copy(1, 1).wait()
        fire(1, 1)

        def body(i, _):
            g = i * 2

            def half(p):
                gg = g + p
                drain_gather(gg, p)

                @pl.when(gg + 2 < n_grp)
                def _():
                    idx_copy(gg + 2, p).start()
                out_copy(gg, p).start()

                @pl.when(gg + 2 < n_grp)
                def _():
                    out_copy(gg, p).wait()
                    idx_copy(gg + 2, p).wait()
                    fire(gg + 2, p)

            half(0)
            half(1)
            return 0

        lax.fori_loop(0, n_grp // 2, body, 0)
        out_copy(n_grp - 2, 0).wait()
        out_copy(n_grp - 1, 1).wait()

    return k(table, idx)


def _sc_scatter_add(m, dst, n_rows):
    E, D = m.shape
    per_w = E // NW
    n_chunks = per_w // CHUNK
    n_slabs = max(k for k in range(1, NS + 1)
                  if n_rows % k == 0 and (n_rows // k) % 8 == 0)
    slab = n_rows // n_slabs
    mesh = plsc.VectorSubcoreMesh(core_axis_name="c", subcore_axis_name="s",
                                  num_cores=NC, num_subcores=NS)
    zeros = jnp.zeros((n_rows, D), jnp.float32)

    dstr = dst.reshape(NW, n_chunks, CHUNK)

    @functools.partial(
        pl.kernel, mesh=mesh,
        out_type=jax.ShapeDtypeStruct((NC, n_rows, D), jnp.float32),
        scratch_types=[
            pltpu.MemorySpace.VMEM_SHARED((n_rows, D), jnp.float32),
            pltpu.VMEM((n_chunks, CHUNK), jnp.int32),
            pltpu.VMEM((CHUNK, D), jnp.float32),
            pltpu.VMEM((CHUNK, D), jnp.float32),
            pltpu.SemaphoreType.DMA,
            pltpu.SemaphoreType.DMA,
        ],
    )
    def k(m_hbm, dst_hbm, z_hbm, parts_hbm, agg_sh, idx_v, rows0, rows1,
          sm0, sm1):
        cid = lax.axis_index("c")
        sid = lax.axis_index("s")
        wid = cid * NS + sid
        row0 = sid * slab
        rows = (rows0, rows1)
        sm = (sm0, sm1)

        pltpu.sync_copy(dst_hbm.at[wid], idx_v)

        @pl.when(sid < n_slabs)
        def _():
            pltpu.sync_copy(z_hbm.at[pl.ds(row0, slab)],
                            agg_sh.at[pl.ds(row0, slab)])
        plsc.subcore_barrier()

        def m_copy(t, p):
            return pltpu.make_async_copy(
                m_hbm.at[pl.ds(wid * per_w + t * CHUNK, CHUNK)],
                rows[p], sm[p])

        m_copy(0, 0).start()
        m_copy(1, 1).start()

        def body(i, _):
            t = i * 2

            def half(p):
                tt = t + p
                m_copy(tt, p).wait()
                pltpu.sync_copy(rows[p], agg_sh.at[idx_v.at[tt]], add=True)

                @pl.when(tt + 2 < n_chunks)
                def _():
                    m_copy(tt + 2, p).start()

            half(0)
            half(1)
            return 0

        lax.fori_loop(0, n_chunks // 2, body, 0)
        if n_chunks % 2 == 1:
            t_last = n_chunks - 1
            m_copy(t_last, 0).wait()
            pltpu.sync_copy(rows[0], agg_sh.at[idx_v.at[t_last]], add=True)
        plsc.subcore_barrier()

        @pl.when(sid < n_slabs)
        def _():
            pltpu.sync_copy(agg_sh.at[pl.ds(row0, slab)],
                            parts_hbm.at[cid, pl.ds(row0, slab)])

    return k(m, dstr, zeros)


def _col(a, j):
    return a[:, j:j + 1]


def _tc_prep(xg, sh, ends, E, Eb):
    nb = E // Eb

    def fn(xs_ref, xd_ref, sh_ref, ends_ref, ef_ref):
        xs = xs_ref[...]
        xd = xd_ref[...]
        s = sh_ref[...]
        d = (xd - jnp.floor(xd)) - (xs - jnp.floor(xs)) + s
        src_f = _col(s, 3)
        bf = jnp.sum((src_f >= ends_ref[...]).astype(jnp.float32), axis=1,
                     keepdims=True)
        ef_ref[...] = jnp.concatenate(
            [d[:, 0:3], bf, jnp.zeros((xs.shape[0], 12), jnp.float32)], axis=1)

    return pl.pallas_call(
        fn,
        grid=(nb,),
        in_specs=[
            pl.BlockSpec((Eb, 16), lambda i: (i, 0)),
            pl.BlockSpec((Eb, 16), lambda i: (nb + i, 0)),
            pl.BlockSpec((Eb, 16), lambda i: (i, 0)),
            pl.BlockSpec((1, 16), lambda i: (0, 0)),
        ],
        out_specs=pl.BlockSpec((Eb, 16), lambda i: (i, 0)),
        out_shape=jax.ShapeDtypeStruct((E, 16), jnp.float32),
    )(xg, xg, sh, ends)


def _vec_cols(ef, rho16):
    bf = _col(ef, 3)
    iot = lax.broadcasted_iota(jnp.int32, (ef.shape[0], 16), 1)
    onehot = (bf == iot.astype(jnp.float32))
    P = _dot(onehot.astype(jnp.float32), rho16)
    v = []
    for j in range(3):
        v.append(_col(ef, 0) * _col(P, 0 + j) + _col(ef, 1) * _col(P, 3 + j)
                 + _col(ef, 2) * _col(P, 6 + j))
    return v, onehot


def _tc_message(g, ef, rho16, w1c, E, Eb, a_off):
    nb = E // Eb

    def fn(as_ref, bd_ref, ef_ref, rho_ref, w1c_ref, m_ref):
        ef_b = ef_ref[...]
        v, _ = _vec_cols(ef_b, rho_ref[...])
        vw = (v[0] * w1c_ref[0:1, :] + v[1] * w1c_ref[1:2, :]
              + v[2] * w1c_ref[2:3, :])
        m_ref[...] = jnp.maximum(as_ref[...] + bd_ref[...] + vw, 0.0)

    return pl.pallas_call(
        fn,
        grid=(nb,),
        in_specs=[
            pl.BlockSpec((Eb, 128), lambda i: (a_off + i, 0)),
            pl.BlockSpec((Eb, 128), lambda i: (a_off + nb + i, 0)),
            pl.BlockSpec((Eb, 16), lambda i: (i, 0)),
            pl.BlockSpec((16, 16), lambda i: (0, 0)),
            pl.BlockSpec((8, 128), lambda i: (0, 0)),
        ],
        out_specs=pl.BlockSpec((Eb, 128), lambda i: (i, 0)),
        out_shape=jax.ShapeDtypeStruct((E, 128), jnp.float32),
    )(g, g, ef, rho16, w1c)


def _tc_update(parts, h, W2, b2p, ab_ws, N, Nb):
    nb = N // Nb
    n_ab = len(ab_ws)

    def fn(*refs):
        parts_ref, h_ref = refs[0], refs[1]
        W2_ref, b2_ref = refs[2], refs[3]
        wrefs = refs[4:4 + 3 * n_ab]
        h_out = refs[4 + 3 * n_ab]
        ab_out = refs[5 + 3 * n_ab]
        agg = parts_ref[0] + parts_ref[1]
        hn = h_ref[...] + jnp.maximum(_dot(agg, W2_ref[...])
                                      + b2_ref[0:1, :], 0.0)
        h_out[...] = hn
        for j in range(n_ab):
            wa, wb, bp = wrefs[3 * j], wrefs[3 * j + 1], wrefs[3 * j + 2]
            ab_out[2 * j] = _dot(hn, wa[...]) + bp[0:1, :]
            ab_out[2 * j + 1] = _dot(hn, wb[...])

    in_specs = [
        pl.BlockSpec((2, Nb, 128), lambda i: (0, i, 0)),
        pl.BlockSpec((Nb, 128), lambda i: (i, 0)),
        pl.BlockSpec((128, 128), lambda i: (0, 0)),
        pl.BlockSpec((8, 128), lambda i: (0, 0)),
    ]
    args = [parts, h, W2, b2p]
    for (wa, wb, bp) in ab_ws:
        in_specs += [pl.BlockSpec((128, 128), lambda i: (0, 0)),
                     pl.BlockSpec((128, 128), lambda i: (0, 0)),
                     pl.BlockSpec((8, 128), lambda i: (0, 0))]
        args += [wa, wb, bp]
    return pl.pallas_call(
        fn,
        grid=(nb,),
        in_specs=in_specs,
        out_specs=[pl.BlockSpec((Nb, 128), lambda i: (i, 0)),
                   pl.BlockSpec((2 * n_ab, Nb, 128), lambda i: (0, i, 0))],
        out_shape=[jax.ShapeDtypeStruct((N, 128), jnp.float32),
                   jax.ShapeDtypeStruct((2 * n_ab, N, 128), jnp.float32)],
    )(*args)


def _tc_ab0(h, wa, wb, bp, N, Nb):
    nb = N // Nb

    def fn(h_ref, wa_ref, wb_ref, bp_ref, ab_ref):
        hb = h_ref[...]
        ab_ref[0] = _dot(hb, wa_ref[...]) + bp_ref[0:1, :]
        ab_ref[1] = _dot(hb, wb_ref[...])

    return pl.pallas_call(
        fn,
        grid=(nb,),
        in_specs=[
            pl.BlockSpec((Nb, 128), lambda i: (i, 0)),
            pl.BlockSpec((128, 128), lambda i: (0, 0)),
            pl.BlockSpec((128, 128), lambda i: (0, 0)),
            pl.BlockSpec((8, 128), lambda i: (0, 0)),
        ],
        out_specs=pl.BlockSpec((2, Nb, 128), lambda i: (0, i, 0)),
        out_shape=jax.ShapeDtypeStruct((2, N, 128), jnp.float32),
    )(h, wa, wb, bp)


def _tc_actions(g, ef, rho16, wa1c, wa2p, ba2p, E, Eb, a_off):
    nb = E // Eb

    def fn(as_ref, bd_ref, ef_ref, rho_ref, wc_ref, wa2_ref, ba2_ref, acc_ref):
        i = pl.program_id(0)
        ef_b = ef_ref[...]
        v, onehot = _vec_cols(ef_b, rho_ref[...])
        vw = (v[0] * wc_ref[0:1, :] + v[1] * wc_ref[1:2, :]
              + v[2] * wc_ref[2:3, :])
        e = jnp.maximum(as_ref[...] + bd_ref[...] + vw, 0.0)
        w8 = _dot(e, wa2_ref[...])
        w = jnp.tanh(_col(w8, 0) + ba2_ref[0:1, 0:1])
        nrm = jnp.sqrt(v[0] * v[0] + v[1] * v[1] + v[2] * v[2]) + 1e-8
        u = [vj / nrm for vj in v]
        cols = []
        for a in range(3):
            for b in range(3):
                cols.append(w * u[a] * u[b])
        cols.append(jnp.ones_like(w))
        g16 = jnp.concatenate(
            cols + [jnp.zeros((ef_b.shape[0], 6), jnp.float32)], axis=1)
        seg = jax.lax.dot_general(onehot.astype(jnp.float32), g16,
                                  (((0,), (0,)), ((), ())),
                                  precision=_HIGH,
                                  preferred_element_type=jnp.float32)

        @pl.when(i == 0)
        def _():
            acc_ref[...] = jnp.zeros_like(acc_ref)

        acc_ref[...] += seg

    return pl.pallas_call(
        fn,
        grid=(nb,),
        in_specs=[
            pl.BlockSpec((Eb, 128), lambda i: (a_off + i, 0)),
            pl.BlockSpec((Eb, 128), lambda i: (a_off + nb + i, 0)),
            pl.BlockSpec((Eb, 16), lambda i: (i, 0)),
            pl.BlockSpec((16, 16), lambda i: (0, 0)),
            pl.BlockSpec((8, 128), lambda i: (0, 0)),
            pl.BlockSpec((128, 8), lambda i: (0, 0)),
            pl.BlockSpec((8, 8), lambda i: (0, 0)),
        ],
        out_specs=pl.BlockSpec((16, 16), lambda i: (0, 0)),
        out_shape=jax.ShapeDtypeStruct((16, 16), jnp.float32),
    )(g, g, ef, rho16, wa1c, wa2p, ba2p)


def _tc_compose(acc, ar_prev, cell16, limit):
    def fn(acc_ref, ar_ref, cell_ref, arn_ref, rho_ref):
        acc_b = acc_ref[...]
        ar = ar_ref[...]
        cl = cell_ref[...]
        cnt = _col(acc_b, 9) + 1e-8
        act = [None] * 9
        for i in range(3):
            for j in range(3):
                act[3 * i + j] = (limit * _col(acc_b, 3 * i + j) / cnt
                                  + (1.0 if i == j else 0.0))
        arn = [None] * 9
        for i in range(3):
            for k in range(3):
                arn[3 * i + k] = (act[3 * i] * _col(ar, k)
                                  + act[3 * i + 1] * _col(ar, 3 + k)
                                  + act[3 * i + 2] * _col(ar, 6 + k))
        rho = [None] * 9
        for i in range(3):
            for k in range(3):
                rho[3 * i + k] = (arn[3 * i] * _col(cl, k)
                                  + arn[3 * i + 1] * _col(cl, 3 + k)
                                  + arn[3 * i + 2] * _col(cl, 6 + k))
        z7 = jnp.zeros((16, 7), jnp.float32)
        arn_ref[...] = jnp.concatenate(arn + [z7], axis=1)
        rho_ref[...] = jnp.concatenate(rho + [z7], axis=1)

    return pl.pallas_call(
        fn,
        grid=(1,),
        in_specs=[pl.BlockSpec((16, 16), lambda i: (0, 0))] * 3,
        out_specs=[pl.BlockSpec((16, 16), lambda i: (0, 0))] * 2,
        out_shape=[jax.ShapeDtypeStruct((16, 16), jnp.float32)] * 2,
    )(acc, ar_prev, cell16)


def kernel(cell, x, z, struct_size, edge_index, edge_attr, emb,
           W1_0, b1_0, W2_0, b2_0, W1_1, b1_1, W2_1, b2_1,
           Wu1, bu1, Wu2, bu2, Wa1, ba1, Wa2, ba2):
    N = x.shape[0]
    E = edge_index.shape[1]
    B = cell.shape[0]
    LIMIT = 0.1
    Eb = 2000
    Nb = 1000

    src = edge_index[0].astype(jnp.int32)
    dst = edge_index[1].astype(jnp.int32)
    idx2 = jnp.concatenate([src, dst + N])
    idx2b = idx2 + 2 * N

    xpad = jnp.pad(x.astype(jnp.float32), ((0, 0), (0, 13)))
    xpad2 = jnp.concatenate([xpad, xpad], axis=0)

    sh = jnp.concatenate([
        (edge_attr - 1).astype(jnp.float32),
        src.astype(jnp.float32)[:, None],
        jnp.zeros((E, 12), jnp.float32)], axis=1)

    ends = jnp.cumsum(struct_size).astype(jnp.float32)[None, :]

    zp_len = ((N + NW * CHUNK - 1) // (NW * CHUNK)) * (NW * CHUNK)
    zpad = jnp.pad(z.astype(jnp.int32), (0, zp_len - N))

    def wsplit(W, b):
        wa = W[:128]
        wb = W[128:256]
        wc = jnp.pad(W[256:259], ((0, 5), (0, 0)))
        bp = jnp.broadcast_to(b[None, :], (8, 128))
        return wa, wb, wc, bp

    W1a_0, W1b_0, W1c_0, b1p_0 = wsplit(W1_0, b1_0)
    W1a_1, W1b_1, W1c_1, b1p_1 = wsplit(W1_1, b1_1)
    Wua, Wub, Wuc, bup = wsplit(Wu1, bu1)
    Waa, Wab, Wac, bap = wsplit(Wa1, ba1)
    Wa2p = jnp.pad(Wa2, ((0, 0), (0, 7)))
    ba2p = jnp.broadcast_to(ba2[None, :], (8, 8))
    b2p_0 = jnp.broadcast_to(b2_0[None, :], (8, 128))
    b2p_1 = jnp.broadcast_to(b2_1[None, :], (8, 128))
    bu2p = jnp.broadcast_to(bu2[None, :], (8, 128))

    cell16 = jnp.pad(cell.reshape(B, 9), ((0, 0), (0, 7)))
    eye16 = jnp.pad(jnp.eye(3, dtype=jnp.float32).reshape(1, 9),
                    ((0, 0), (0, 7)))
    eye16 = jnp.broadcast_to(eye16, (16, 16))

    xg = _sc_gather(xpad2, idx2)
    ef = _tc_prep(xg, sh, ends, E, Eb)
    h0 = _sc_gather(emb, zpad)[:N]

    ab = _tc_ab0(h0, W1a_0, W1b_0, b1p_0, N, Nb)
    g = _sc_gather(ab.reshape(2 * N, 128), idx2)
    m = _tc_message(g, ef, cell16, W1c_0, E, Eb, 0)
    parts = _sc_scatter_add(m, dst, N)
    h1, ab = _tc_update(parts, h0, W2_0, b2p_0,
                        [(W1a_1, W1b_1, b1p_1)], N, Nb)

    g = _sc_gather(ab.reshape(2 * N, 128), idx2)
    m = _tc_message(g, ef, cell16, W1c_1, E, Eb, 0)
    parts = _sc_scatter_add(m, dst, N)
    h2, ab = _tc_update(parts, h1, W2_1, b2p_1, [(Wua, Wub, bup)], N, Nb)

    g = _sc_gather(ab.reshape(2 * N, 128), idx2)
    m = _tc_message(g, ef, cell16, Wuc, E, Eb, 0)
    parts = _sc_scatter_add(m, dst, N)
    h3, ab4 = _tc_update(parts, h2, Wu2, bu2p,
                         [(Wua, Wub, bup), (Waa, Wab, bap)], N, Nb)

    ab4r = ab4.reshape(4 * N, 128)
    g4a = _sc_gather(ab4r, idx2)
    g4b = _sc_gather(ab4r, idx2b)
    acc0 = _tc_actions(g4b, ef, cell16, Wac, Wa2p, ba2p, E, Eb, 0)
    ar0, rho1 = _tc_compose(acc0, eye16, cell16, LIMIT)

    m = _tc_message(g4a, ef, rho1, Wuc, E, Eb, 0)
    parts = _sc_scatter_add(m, dst, N)
    h4, ab = _tc_update(parts, h3, Wu2, bu2p, [(Waa, Wab, bap)], N, Nb)

    g = _sc_gather(ab.reshape(2 * N, 128), idx2)
    acc1 = _tc_actions(g, ef, rho1, Wac, Wa2p, ba2p, E, Eb, 0)
    ar1, rho2 = _tc_compose(acc1, ar0, cell16, LIMIT)

    rho1m = rho1[:, :9].reshape(B, 3, 3)
    rho2m = rho2[:, :9].reshape(B, 3, 3)
    ar0m = ar0[:, :9].reshape(B, 3, 3)
    ar1m = ar1[:, :9].reshape(B, 3, 3)
    return (rho2m, jnp.stack([rho1m, rho2m]), jnp.stack([ar0m, ar1m]))
